# baseline (speedup 1.0000x reference)
import functools

import jax
import jax.numpy as jnp
from jax import lax
from jax.experimental import pallas as pl
from jax.experimental.pallas import tpu as pltpu
from jax.experimental.pallas import tpu_sc as plsc

_N = 10000
_E = 320000
_C = 128
_EPS = 1e-5

_NC = 2
_NS = 16
_NW = _NC * _NS
_EPW = _E // _NW
_RPT = _N // _NS
_CH = 48
_NFULL = _EPW // _CH
_REM = _EPW - _NFULL * _CH

_WSTRIDE = 624
_WSIZE = 640


def _mesh():
    return plsc.VectorSubcoreMesh(core_axis_name="c", subcore_axis_name="s")


def _deg_body(ei_hbm, out_hbm, dstv, cnt):
    c = lax.axis_index("c")
    s = lax.axis_index("s")
    gw = c * _NS + s
    zeros16 = jnp.zeros((16,), jnp.float32)

    def zero(i, carry):
        cnt[pl.ds(i * 16, 16)] = zeros16
        return carry

    lax.fori_loop(0, _N // 16, zero, None)
    pltpu.sync_copy(ei_hbm.at[pl.ds(_E + gw * _EPW, _EPW)], dstv)
    ones16 = jnp.ones((16,), jnp.float32)

    def body(i, carry):
        idx = dstv[pl.ds(i * 16, 16)]
        plsc.addupdate_scatter(cnt, [idx], ones16)
        return carry

    lax.fori_loop(0, _EPW // 16, body, None)
    pltpu.sync_copy(cnt, out_hbm.at[pl.ds(gw * _N, _N)])


_deg = pl.kernel(
    _deg_body,
    out_type=jax.ShapeDtypeStruct((_NW * _N,), jnp.float32),
    mesh=_mesh(),
    compiler_params=pltpu.CompilerParams(needs_layout_passes=False),
    scratch_types=[
        pltpu.VMEM((_EPW,), jnp.int32),
        pltpu.VMEM((_N,), jnp.float32),
    ],
)


_R = 4


def _agg_body(ei_hbm, tab_hbm, out_hbm,
              srcall, dstall, rows0, rows1, rows2, rows3,
              dx0, dx1, dx2, dx3, drem, rrem,
              g0, g1, g2, g3, s0, s1, s2, s3, acc):
    rows = (rows0, rows1, rows2, rows3)
    dxs = (dx0, dx1, dx2, dx3)
    gsems = (g0, g1, g2, g3)
    ssems = (s0, s1, s2, s3)
    c = lax.axis_index("c")
    s = lax.axis_index("s")
    gw = c * _NS + s
    base_e = gw * _EPW
    row0 = s * _WSTRIDE

    zeros16 = jnp.zeros((16,), jnp.float32)

    def zrow(i, carry):
        r = i >> 3
        cc = (i & 7) * 16
        rows0[r, pl.ds(cc, 16)] = zeros16
        return carry

    lax.fori_loop(0, _CH * 8, zrow, None)
    nfull = _WSIZE // _CH
    ztail = _WSIZE - nfull * _CH
    for w in range(nfull):
        pltpu.async_copy(rows0, acc.at[pl.ds(row0 + w * _CH, _CH)], s0)
    pltpu.async_copy(rows0.at[pl.ds(0, ztail)],
                     acc.at[pl.ds(row0 + nfull * _CH, ztail)], s0)
    pltpu.async_copy(ei_hbm.at[pl.ds(base_e, _EPW)], srcall, g0)
    pltpu.async_copy(ei_hbm.at[pl.ds(_E + base_e, _EPW)], dstall, g1)
    for w in range(nfull):
        pltpu.make_async_copy(rows0, acc.at[pl.ds(row0 + w * _CH, _CH)],
                              s0).wait()
    pltpu.make_async_copy(rows0.at[pl.ds(0, ztail)],
                          acc.at[pl.ds(row0 + nfull * _CH, ztail)], s0).wait()
    pltpu.make_async_copy(ei_hbm.at[pl.ds(base_e, _EPW)], srcall, g0).wait()
    pltpu.make_async_copy(ei_hbm.at[pl.ds(_E + base_e, _EPW)], dstall,
                          g1).wait()
    plsc.subcore_barrier()

    def start_gather(i, r):
        pltpu.async_copy(tab_hbm.at[srcall.at[pl.ds(i * _CH, _CH)]],
                         rows[r], gsems[r])

    def wait_gather(r):
        pltpu.make_async_copy(tab_hbm.at[srcall.at[pl.ds(0, _CH)]],
                              rows[r], gsems[r]).wait()

    def fill_dx(i, r):
        for jj in range(_CH // 16):
            dxs[r][pl.ds(jj * 16, 16)] = dstall[pl.ds(i * _CH + jj * 16, 16)]

    def start_scatter(r):
        pltpu.async_copy(rows[r], acc.at[dxs[r]], ssems[r], add=True)

    def wait_scatter(r):
        pltpu.make_async_copy(rows[r], acc.at[dxs[r]], ssems[r]).wait()

    for r in range(_R):
        start_gather(r, r)

    def body(k, carry):
        i0 = k * _R
        for r in range(_R):
            wait_gather(r)
            fill_dx(i0 + r, r)
            start_scatter(r)
        for r in range(_R):
            wait_scatter(r)
            start_gather(i0 + _R + r, r)
        return carry

    lax.fori_loop(0, _NFULL // _R - 1, body, None)

    i0 = _NFULL - _R
    for r in range(_R):
        wait_gather(r)
        fill_dx(i0 + r, r)
        start_scatter(r)

    be = _NFULL * _CH
    drem[pl.ds(0, _REM)] = dstall[pl.ds(be, _REM)]
    pltpu.sync_copy(tab_hbm.at[srcall.at[pl.ds(be, _REM)]], rrem)
    pltpu.sync_copy(rrem, acc.at[drem], add=True)

    for r in range(_R):
        wait_scatter(r)

    plsc.subcore_barrier()
    pltpu.sync_copy(acc.at[pl.ds(row0, _WSIZE)],
                    out_hbm.at[c, pl.ds(row0, _WSIZE)])


_agg = pl.kernel(
    _agg_body,
    out_type=jax.ShapeDtypeStruct((_NC, _N, _C), jnp.float32),
    mesh=_mesh(),
    compiler_params=pltpu.CompilerParams(needs_layout_passes=False),
    scratch_types=[
        pltpu.VMEM((_EPW,), jnp.int32),
        pltpu.VMEM((_EPW,), jnp.int32),
        pltpu.VMEM((_CH, _C), jnp.float32),
        pltpu.VMEM((_CH, _C), jnp.float32),
        pltpu.VMEM((_CH, _C), jnp.float32),
        pltpu.VMEM((_CH, _C), jnp.float32),
        pltpu.VMEM((_CH,), jnp.int32),
        pltpu.VMEM((_CH,), jnp.int32),
        pltpu.VMEM((_CH,), jnp.int32),
        pltpu.VMEM((_CH,), jnp.int32),
        pltpu.VMEM((_REM,), jnp.int32),
        pltpu.VMEM((_REM, _C), jnp.float32),
        pltpu.SemaphoreType.DMA,
        pltpu.SemaphoreType.DMA,
        pltpu.SemaphoreType.DMA,
        pltpu.SemaphoreType.DMA,
        pltpu.SemaphoreType.DMA,
        pltpu.SemaphoreType.DMA,
        pltpu.SemaphoreType.DMA,
        pltpu.SemaphoreType.DMA,
        pltpu.VMEM_SHARED((_N, _C), jnp.float32),
    ],
)


def _bn(v, w, b):
    m = jnp.mean(v, axis=0, keepdims=True)
    d = v - m
    var = jnp.mean(d * d, axis=0, keepdims=True)
    return d * lax.rsqrt(var + _EPS) * w + b


def _k2_body(x_ref, w0_ref, b0_ref, W1_ref, degp_ref, y_ref, dis_ref):
    xn = _bn(x_ref[...], w0_ref[...], b0_ref[...])
    xw = jnp.dot(xn, W1_ref[...], preferred_element_type=jnp.float32)
    ones = jnp.ones((_NW, 1), jnp.float32)
    deg = lax.dot_general(degp_ref[...], ones,
                          (((0,), (0,)), ((), ())),
                          preferred_element_type=jnp.float32) + 1.0
    dis = lax.rsqrt(deg)
    y_ref[...] = xw * dis
    dis_ref[...] = dis


_k2 = pl.pallas_call(
    _k2_body,
    out_shape=(
        jax.ShapeDtypeStruct((_N, _C), jnp.float32),
        jax.ShapeDtypeStruct((_N, 1), jnp.float32),
    ),
)


def _k4_body(accp_ref, y_ref, dis_ref, b1_ref, w1_ref, bb1_ref, Wroot_ref,
             h_ref, hr_ref):
    acc = accp_ref[0] + accp_ref[1]
    g = dis_ref[...] * (acc + y_ref[...]) + b1_ref[...]
    g = jnp.maximum(g, 0.0)
    h = _bn(g, w1_ref[...], bb1_ref[...])
    h_ref[...] = h
    hr_ref[...] = jnp.dot(h, Wroot_ref[...],
                          preferred_element_type=jnp.float32)


_k4 = pl.pallas_call(
    _k4_body,
    out_shape=(
        jax.ShapeDtypeStruct((_N, _C), jnp.float32),
        jax.ShapeDtypeStruct((_N, _C), jnp.float32),
    ),
)


def _k6b_body(accp_ref, hr_ref, Wrel_ref, b2_ref, w2_ref, bb2_ref, o_ref):
    acc = accp_ref[0] + accp_ref[1]
    z = (jnp.dot(acc, Wrel_ref[...], preferred_element_type=jnp.float32)
         + b2_ref[...] + hr_ref[...])
    z = jnp.maximum(z, 0.0)
    o_ref[...] = _bn(z, w2_ref[...], bb2_ref[...])


_k6b = pl.pallas_call(
    _k6b_body,
    out_shape=jax.ShapeDtypeStruct((_N, _C), jnp.float32),
)


def kernel(x, edge_index, bn0_w, bn0_b, gcn1_W, gcn1_b, bn1_w, bn1_b,
           gc2_W_rel, gc2_W_root, gc2_b, bn2_w, bn2_b):
    ei = edge_index.astype(jnp.int32).reshape(2 * _E)

    degp = _deg(ei).reshape(_NW, _N)
    y, dis = _k2(x, bn0_w.reshape(1, _C), bn0_b.reshape(1, _C), gcn1_W, degp)
    accp = _agg(ei, y)
    h, hroot = _k4(accp, y, dis, gcn1_b.reshape(1, _C), bn1_w.reshape(1, _C),
                   bn1_b.reshape(1, _C), gc2_W_root)
    acc2p = _agg(ei, h)
    out = _k6b(acc2p, hroot, gc2_W_rel, gc2_b.reshape(1, _C),
               bn2_w.reshape(1, _C), bn2_b.reshape(1, _C))
    return out


# speedup vs baseline: 1.0043x; 1.0043x over previous
import functools

import jax
import jax.numpy as jnp
from jax import lax
from jax.experimental import pallas as pl
from jax.experimental.pallas import tpu as pltpu
from jax.experimental.pallas import tpu_sc as plsc

_N = 10000
_E = 320000
_C = 128
_EPS = 1e-5

_NC = 2
_NS = 16
_NW = _NC * _NS
_EPW = _E // _NW
_RPT = _N // _NS
_CH = 48
_NFULL = _EPW // _CH
_REM = _EPW - _NFULL * _CH

_WSTRIDE = 624
_WSIZE = 640


def _mesh():
    return plsc.VectorSubcoreMesh(core_axis_name="c", subcore_axis_name="s")


def _deg_body(ei_hbm, out_hbm, dstv, cnt):
    c = lax.axis_index("c")
    s = lax.axis_index("s")
    gw = c * _NS + s
    zeros16 = jnp.zeros((16,), jnp.float32)

    def zero(i, carry):
        cnt[pl.ds(i * 16, 16)] = zeros16
        return carry

    lax.fori_loop(0, _N // 16, zero, None)
    pltpu.sync_copy(ei_hbm.at[pl.ds(_E + gw * _EPW, _EPW)], dstv)
    ones16 = jnp.ones((16,), jnp.float32)

    def body(i, carry):
        idx = dstv[pl.ds(i * 16, 16)]
        plsc.addupdate_scatter(cnt, [idx], ones16)
        return carry

    lax.fori_loop(0, _EPW // 16, body, None)
    pltpu.sync_copy(cnt, out_hbm.at[pl.ds(gw * _N, _N)])


_deg = pl.kernel(
    _deg_body,
    out_type=jax.ShapeDtypeStruct((_NW * _N,), jnp.float32),
    mesh=_mesh(),
    compiler_params=pltpu.CompilerParams(needs_layout_passes=False),
    scratch_types=[
        pltpu.VMEM((_EPW,), jnp.int32),
        pltpu.VMEM((_N,), jnp.float32),
    ],
)


_R = 4


def _agg_body(ei_hbm, tab_hbm, out_hbm,
              srcall, dstall, rows0, rows1, rows2, rows3,
              dx0, dx1, dx2, dx3, drem, rrem,
              g0, g1, g2, g3, s0, s1, s2, s3, acc):
    rows = (rows0, rows1, rows2, rows3)
    dxs = (dx0, dx1, dx2, dx3)
    gsems = (g0, g1, g2, g3)
    ssems = (s0, s1, s2, s3)
    c = lax.axis_index("c")
    s = lax.axis_index("s")
    gw = c * _NS + s
    base_e = gw * _EPW
    row0 = s * _WSTRIDE

    zeros16 = jnp.zeros((16,), jnp.float32)

    def zrow(i, carry):
        r = i >> 3
        cc = (i & 7) * 16
        rows0[r, pl.ds(cc, 16)] = zeros16
        return carry

    lax.fori_loop(0, _CH * 8, zrow, None)
    nfull = _WSIZE // _CH
    ztail = _WSIZE - nfull * _CH
    for w in range(nfull):
        pltpu.async_copy(rows0, acc.at[pl.ds(row0 + w * _CH, _CH)], s0)
    pltpu.async_copy(rows0.at[pl.ds(0, ztail)],
                     acc.at[pl.ds(row0 + nfull * _CH, ztail)], s0)
    pltpu.async_copy(ei_hbm.at[pl.ds(base_e, _EPW)], srcall, g0)
    pltpu.async_copy(ei_hbm.at[pl.ds(_E + base_e, _EPW)], dstall, g1)
    for w in range(nfull):
        pltpu.make_async_copy(rows0, acc.at[pl.ds(row0 + w * _CH, _CH)],
                              s0).wait()
    pltpu.make_async_copy(rows0.at[pl.ds(0, ztail)],
                          acc.at[pl.ds(row0 + nfull * _CH, ztail)], s0).wait()
    pltpu.make_async_copy(ei_hbm.at[pl.ds(base_e, _EPW)], srcall, g0).wait()
    pltpu.make_async_copy(ei_hbm.at[pl.ds(_E + base_e, _EPW)], dstall,
                          g1).wait()
    plsc.subcore_barrier()

    def start_gather(i, r):
        pltpu.async_copy(tab_hbm.at[srcall.at[pl.ds(i * _CH, _CH)]],
                         rows[r], gsems[r])

    def wait_gather(r):
        pltpu.make_async_copy(tab_hbm.at[srcall.at[pl.ds(0, _CH)]],
                              rows[r], gsems[r]).wait()

    def fill_dx(i, r):
        for jj in range(_CH // 16):
            dxs[r][pl.ds(jj * 16, 16)] = dstall[pl.ds(i * _CH + jj * 16, 16)]

    def start_scatter(r):
        pltpu.async_copy(rows[r], acc.at[dxs[r]], ssems[r], add=True)

    def wait_scatter(r):
        pltpu.make_async_copy(rows[r], acc.at[dxs[r]], ssems[r]).wait()

    for r in range(_R):
        start_gather(r, r)

    def body(k, carry):
        i0 = k * _R
        for r in range(_R):
            wait_gather(r)
            fill_dx(i0 + r, r)
            start_scatter(r)
        for r in range(_R):
            wait_scatter(r)
            start_gather(i0 + _R + r, r)
        return carry

    lax.fori_loop(0, _NFULL // _R - 1, body, None)

    i0 = _NFULL - _R
    for r in range(_R):
        wait_gather(r)
        fill_dx(i0 + r, r)
        start_scatter(r)

    be = _NFULL * _CH
    drem[pl.ds(0, _REM)] = dstall[pl.ds(be, _REM)]
    pltpu.sync_copy(tab_hbm.at[srcall.at[pl.ds(be, _REM)]], rrem)
    pltpu.sync_copy(rrem, acc.at[drem], add=True)

    for r in range(_R):
        wait_scatter(r)

    plsc.subcore_barrier()
    pltpu.sync_copy(acc.at[pl.ds(row0, _WSIZE)],
                    out_hbm.at[c, pl.ds(row0, _WSIZE)])


_agg = pl.kernel(
    _agg_body,
    out_type=jax.ShapeDtypeStruct((_NC, _N, _C), jnp.float32),
    mesh=_mesh(),
    compiler_params=pltpu.CompilerParams(needs_layout_passes=False),
    scratch_types=[
        pltpu.VMEM((_EPW,), jnp.int32),
        pltpu.VMEM((_EPW,), jnp.int32),
        pltpu.VMEM((_CH, _C), jnp.float32),
        pltpu.VMEM((_CH, _C), jnp.float32),
        pltpu.VMEM((_CH, _C), jnp.float32),
        pltpu.VMEM((_CH, _C), jnp.float32),
        pltpu.VMEM((_CH,), jnp.int32),
        pltpu.VMEM((_CH,), jnp.int32),
        pltpu.VMEM((_CH,), jnp.int32),
        pltpu.VMEM((_CH,), jnp.int32),
        pltpu.VMEM((_REM,), jnp.int32),
        pltpu.VMEM((_REM, _C), jnp.float32),
        pltpu.SemaphoreType.DMA,
        pltpu.SemaphoreType.DMA,
        pltpu.SemaphoreType.DMA,
        pltpu.SemaphoreType.DMA,
        pltpu.SemaphoreType.DMA,
        pltpu.SemaphoreType.DMA,
        pltpu.SemaphoreType.DMA,
        pltpu.SemaphoreType.DMA,
        pltpu.VMEM_SHARED((_N, _C), jnp.float32),
    ],
)


def _bn(v, w, b):
    m = jnp.mean(v, axis=0, keepdims=True)
    d = v - m
    var = jnp.mean(d * d, axis=0, keepdims=True)
    return d * lax.rsqrt(var + _EPS) * w + b


def _k2a_body(x_ref, w0_ref, b0_ref, W1_ref, xw_ref):
    xn = _bn(x_ref[...], w0_ref[...], b0_ref[...])
    xw_ref[...] = jnp.dot(xn, W1_ref[...], preferred_element_type=jnp.float32)


_k2a = pl.pallas_call(
    _k2a_body,
    out_shape=jax.ShapeDtypeStruct((_N, _C), jnp.float32),
)


def _k2b_body(xw_ref, degp_ref, y_ref, dis_ref):
    ones = jnp.ones((_NW, 1), jnp.float32)
    deg = lax.dot_general(degp_ref[...], ones,
                          (((0,), (0,)), ((), ())),
                          preferred_element_type=jnp.float32) + 1.0
    dis = lax.rsqrt(deg)
    y_ref[...] = xw_ref[...] * dis
    dis_ref[...] = dis


_k2b = pl.pallas_call(
    _k2b_body,
    out_shape=(
        jax.ShapeDtypeStruct((_N, _C), jnp.float32),
        jax.ShapeDtypeStruct((_N, 1), jnp.float32),
    ),
)


def _k4_body(accp_ref, y_ref, dis_ref, b1_ref, w1_ref, bb1_ref, Wroot_ref,
             h_ref, hr_ref):
    acc = accp_ref[0] + accp_ref[1]
    g = dis_ref[...] * (acc + y_ref[...]) + b1_ref[...]
    g = jnp.maximum(g, 0.0)
    h = _bn(g, w1_ref[...], bb1_ref[...])
    h_ref[...] = h
    hr_ref[...] = jnp.dot(h, Wroot_ref[...],
                          preferred_element_type=jnp.float32)


_k4 = pl.pallas_call(
    _k4_body,
    out_shape=(
        jax.ShapeDtypeStruct((_N, _C), jnp.float32),
        jax.ShapeDtypeStruct((_N, _C), jnp.float32),
    ),
)


def _k6b_body(accp_ref, hr_ref, Wrel_ref, b2_ref, w2_ref, bb2_ref, o_ref):
    acc = accp_ref[0] + accp_ref[1]
    z = (jnp.dot(acc, Wrel_ref[...], preferred_element_type=jnp.float32)
         + b2_ref[...] + hr_ref[...])
    z = jnp.maximum(z, 0.0)
    o_ref[...] = _bn(z, w2_ref[...], bb2_ref[...])


_k6b = pl.pallas_call(
    _k6b_body,
    out_shape=jax.ShapeDtypeStruct((_N, _C), jnp.float32),
)


def kernel(x, edge_index, bn0_w, bn0_b, gcn1_W, gcn1_b, bn1_w, bn1_b,
           gc2_W_rel, gc2_W_root, gc2_b, bn2_w, bn2_b):
    ei = edge_index.astype(jnp.int32).reshape(2 * _E)

    degp = _deg(ei).reshape(_NW, _N)
    xw = _k2a(x, bn0_w.reshape(1, _C), bn0_b.reshape(1, _C), gcn1_W)
    y, dis = _k2b(xw, degp)
    accp = _agg(ei, y)
    h, hroot = _k4(accp, y, dis, gcn1_b.reshape(1, _C), bn1_w.reshape(1, _C),
                   bn1_b.reshape(1, _C), gc2_W_root)
    acc2p = _agg(ei, h)
    out = _k6b(acc2p, hroot, gc2_W_rel, gc2_b.reshape(1, _C),
               bn2_w.reshape(1, _C), bn2_b.reshape(1, _C))
    return out


# speedup vs baseline: 1.0088x; 1.0045x over previous
import functools

import jax
import jax.numpy as jnp
from jax import lax
from jax.experimental import pallas as pl
from jax.experimental.pallas import tpu as pltpu
from jax.experimental.pallas import tpu_sc as plsc

_N = 10000
_E = 320000
_C = 128
_EPS = 1e-5

_NC = 2
_NS = 16
_NW = _NC * _NS
_EPW = _E // _NW
_RPT = _N // _NS
_CH = 48
_NFULL = _EPW // _CH
_REM = _EPW - _NFULL * _CH

_WSTRIDE = 624
_WSIZE = 640


def _mesh():
    return plsc.VectorSubcoreMesh(core_axis_name="c", subcore_axis_name="s")


def _deg_body(ei_hbm, out_hbm, dstv, cnt):
    c = lax.axis_index("c")
    s = lax.axis_index("s")
    gw = c * _NS + s
    zeros16 = jnp.zeros((16,), jnp.float32)
    _U = 5

    def zero(i, carry):
        for u in range(_U):
            cnt[pl.ds((i * _U + u) * 16, 16)] = zeros16
        return carry

    lax.fori_loop(0, _N // 16 // _U, zero, None)
    pltpu.sync_copy(ei_hbm.at[pl.ds(_E + gw * _EPW, _EPW)], dstv)
    ones16 = jnp.ones((16,), jnp.float32)

    def body(i, carry):
        for u in range(_U):
            idx = dstv[pl.ds((i * _U + u) * 16, 16)]
            plsc.addupdate_scatter(cnt, [idx], ones16)
        return carry

    lax.fori_loop(0, _EPW // 16 // _U, body, None)
    pltpu.sync_copy(cnt, out_hbm.at[pl.ds(gw * _N, _N)])


_deg = pl.kernel(
    _deg_body,
    out_type=jax.ShapeDtypeStruct((_NW * _N,), jnp.float32),
    mesh=_mesh(),
    compiler_params=pltpu.CompilerParams(needs_layout_passes=False),
    scratch_types=[
        pltpu.VMEM((_EPW,), jnp.int32),
        pltpu.VMEM((_N,), jnp.float32),
    ],
)


_R = 4


def _agg_body(ei_hbm, tab_hbm, out_hbm,
              srcall, dstall, rows0, rows1, rows2, rows3,
              dx0, dx1, dx2, dx3, drem, rrem,
              g0, g1, g2, g3, s0, s1, s2, s3, acc):
    rows = (rows0, rows1, rows2, rows3)
    dxs = (dx0, dx1, dx2, dx3)
    gsems = (g0, g1, g2, g3)
    ssems = (s0, s1, s2, s3)
    c = lax.axis_index("c")
    s = lax.axis_index("s")
    gw = c * _NS + s
    base_e = gw * _EPW
    row0 = s * _WSTRIDE

    zeros16 = jnp.zeros((16,), jnp.float32)

    def zrow(i, carry):
        r = i >> 3
        cc = (i & 7) * 16
        rows0[r, pl.ds(cc, 16)] = zeros16
        return carry

    lax.fori_loop(0, _CH * 8, zrow, None)
    nfull = _WSIZE // _CH
    ztail = _WSIZE - nfull * _CH
    for w in range(nfull):
        pltpu.async_copy(rows0, acc.at[pl.ds(row0 + w * _CH, _CH)], s0)
    pltpu.async_copy(rows0.at[pl.ds(0, ztail)],
                     acc.at[pl.ds(row0 + nfull * _CH, ztail)], s0)
    pltpu.async_copy(ei_hbm.at[pl.ds(base_e, _EPW)], srcall, g0)
    pltpu.async_copy(ei_hbm.at[pl.ds(_E + base_e, _EPW)], dstall, g1)
    for w in range(nfull):
        pltpu.make_async_copy(rows0, acc.at[pl.ds(row0 + w * _CH, _CH)],
                              s0).wait()
    pltpu.make_async_copy(rows0.at[pl.ds(0, ztail)],
                          acc.at[pl.ds(row0 + nfull * _CH, ztail)], s0).wait()
    pltpu.make_async_copy(ei_hbm.at[pl.ds(base_e, _EPW)], srcall, g0).wait()
    pltpu.make_async_copy(ei_hbm.at[pl.ds(_E + base_e, _EPW)], dstall,
                          g1).wait()
    plsc.subcore_barrier()

    def start_gather(i, r):
        pltpu.async_copy(tab_hbm.at[srcall.at[pl.ds(i * _CH, _CH)]],
                         rows[r], gsems[r])

    def wait_gather(r):
        pltpu.make_async_copy(tab_hbm.at[srcall.at[pl.ds(0, _CH)]],
                              rows[r], gsems[r]).wait()

    def fill_dx(i, r):
        for jj in range(_CH // 16):
            dxs[r][pl.ds(jj * 16, 16)] = dstall[pl.ds(i * _CH + jj * 16, 16)]

    def start_scatter(r):
        pltpu.async_copy(rows[r], acc.at[dxs[r]], ssems[r], add=True)

    def wait_scatter(r):
        pltpu.make_async_copy(rows[r], acc.at[dxs[r]], ssems[r]).wait()

    for r in range(_R):
        start_gather(r, r)

    def body(k, carry):
        i0 = k * _R
        for r in range(_R):
            wait_gather(r)
            fill_dx(i0 + r, r)
            start_scatter(r)
        for r in range(_R):
            wait_scatter(r)
            start_gather(i0 + _R + r, r)
        return carry

    lax.fori_loop(0, _NFULL // _R - 1, body, None)

    i0 = _NFULL - _R
    for r in range(_R):
        wait_gather(r)
        fill_dx(i0 + r, r)
        start_scatter(r)

    be = _NFULL * _CH
    drem[pl.ds(0, _REM)] = dstall[pl.ds(be, _REM)]
    pltpu.sync_copy(tab_hbm.at[srcall.at[pl.ds(be, _REM)]], rrem)
    pltpu.sync_copy(rrem, acc.at[drem], add=True)

    for r in range(_R):
        wait_scatter(r)

    plsc.subcore_barrier()
    pltpu.sync_copy(acc.at[pl.ds(row0, _WSIZE)],
                    out_hbm.at[c, pl.ds(row0, _WSIZE)])


_agg = pl.kernel(
    _agg_body,
    out_type=jax.ShapeDtypeStruct((_NC, _N, _C), jnp.float32),
    mesh=_mesh(),
    compiler_params=pltpu.CompilerParams(needs_layout_passes=False),
    scratch_types=[
        pltpu.VMEM((_EPW,), jnp.int32),
        pltpu.VMEM((_EPW,), jnp.int32),
        pltpu.VMEM((_CH, _C), jnp.float32),
        pltpu.VMEM((_CH, _C), jnp.float32),
        pltpu.VMEM((_CH, _C), jnp.float32),
        pltpu.VMEM((_CH, _C), jnp.float32),
        pltpu.VMEM((_CH,), jnp.int32),
        pltpu.VMEM((_CH,), jnp.int32),
        pltpu.VMEM((_CH,), jnp.int32),
        pltpu.VMEM((_CH,), jnp.int32),
        pltpu.VMEM((_REM,), jnp.int32),
        pltpu.VMEM((_REM, _C), jnp.float32),
        pltpu.SemaphoreType.DMA,
        pltpu.SemaphoreType.DMA,
        pltpu.SemaphoreType.DMA,
        pltpu.SemaphoreType.DMA,
        pltpu.SemaphoreType.DMA,
        pltpu.SemaphoreType.DMA,
        pltpu.SemaphoreType.DMA,
        pltpu.SemaphoreType.DMA,
        pltpu.VMEM_SHARED((_N, _C), jnp.float32),
    ],
)


def _bn(v, w, b):
    m = jnp.mean(v, axis=0, keepdims=True)
    d = v - m
    var = jnp.mean(d * d, axis=0, keepdims=True)
    return d * lax.rsqrt(var + _EPS) * w + b


def _k2a_body(x_ref, w0_ref, b0_ref, W1_ref, xw_ref):
    xn = _bn(x_ref[...], w0_ref[...], b0_ref[...])
    xw_ref[...] = jnp.dot(xn, W1_ref[...], preferred_element_type=jnp.float32)


_k2a = pl.pallas_call(
    _k2a_body,
    out_shape=jax.ShapeDtypeStruct((_N, _C), jnp.float32),
)


def _k2b_body(xw_ref, degp_ref, y_ref, dis_ref):
    ones = jnp.ones((_NW, 1), jnp.float32)
    deg = lax.dot_general(degp_ref[...], ones,
                          (((0,), (0,)), ((), ())),
                          preferred_element_type=jnp.float32) + 1.0
    dis = lax.rsqrt(deg)
    y_ref[...] = xw_ref[...] * dis
    dis_ref[...] = dis


_k2b = pl.pallas_call(
    _k2b_body,
    out_shape=(
        jax.ShapeDtypeStruct((_N, _C), jnp.float32),
        jax.ShapeDtypeStruct((_N, 1), jnp.float32),
    ),
)


def _k4_body(accp_ref, y_ref, dis_ref, b1_ref, w1_ref, bb1_ref, Wroot_ref,
             h_ref, hr_ref):
    acc = accp_ref[0] + accp_ref[1]
    g = dis_ref[...] * (acc + y_ref[...]) + b1_ref[...]
    g = jnp.maximum(g, 0.0)
    h = _bn(g, w1_ref[...], bb1_ref[...])
    h_ref[...] = h
    hr_ref[...] = jnp.dot(h, Wroot_ref[...],
                          preferred_element_type=jnp.float32)


_k4 = pl.pallas_call(
    _k4_body,
    out_shape=(
        jax.ShapeDtypeStruct((_N, _C), jnp.float32),
        jax.ShapeDtypeStruct((_N, _C), jnp.float32),
    ),
)


def _k6b_body(accp_ref, hr_ref, Wrel_ref, b2_ref, w2_ref, bb2_ref, o_ref):
    acc = accp_ref[0] + accp_ref[1]
    z = (jnp.dot(acc, Wrel_ref[...], preferred_element_type=jnp.float32)
         + b2_ref[...] + hr_ref[...])
    z = jnp.maximum(z, 0.0)
    o_ref[...] = _bn(z, w2_ref[...], bb2_ref[...])


_k6b = pl.pallas_call(
    _k6b_body,
    out_shape=jax.ShapeDtypeStruct((_N, _C), jnp.float32),
)


def kernel(x, edge_index, bn0_w, bn0_b, gcn1_W, gcn1_b, bn1_w, bn1_b,
           gc2_W_rel, gc2_W_root, gc2_b, bn2_w, bn2_b):
    ei = edge_index.astype(jnp.int32).reshape(2 * _E)

    degp = _deg(ei).reshape(_NW, _N)
    xw = _k2a(x, bn0_w.reshape(1, _C), bn0_b.reshape(1, _C), gcn1_W)
    y, dis = _k2b(xw, degp)
    accp = _agg(ei, y)
    h, hroot = _k4(accp, y, dis, gcn1_b.reshape(1, _C), bn1_w.reshape(1, _C),
                   bn1_b.reshape(1, _C), gc2_W_root)
    acc2p = _agg(ei, h)
    out = _k6b(acc2p, hroot, gc2_W_rel, gc2_b.reshape(1, _C),
               bn2_w.reshape(1, _C), bn2_b.reshape(1, _C))
    return out


# speedup vs baseline: 1.0235x; 1.0145x over previous
import functools

import jax
import jax.numpy as jnp
from jax import lax
from jax.experimental import pallas as pl
from jax.experimental.pallas import tpu as pltpu
from jax.experimental.pallas import tpu_sc as plsc

_N = 10000
_E = 320000
_C = 128
_EPS = 1e-5

_NC = 2
_NS = 16
_NW = _NC * _NS
_EPW = _E // _NW
_RPT = _N // _NS
_CH = 64
_NFULL = _EPW // _CH
_REM = _EPW - _NFULL * _CH

_WSTRIDE = 624
_WSIZE = 640


def _mesh():
    return plsc.VectorSubcoreMesh(core_axis_name="c", subcore_axis_name="s")


def _deg_body(ei_hbm, out_hbm, dstv, cnt):
    c = lax.axis_index("c")
    s = lax.axis_index("s")
    gw = c * _NS + s
    zeros16 = jnp.zeros((16,), jnp.float32)
    _U = 5

    def zero(i, carry):
        for u in range(_U):
            cnt[pl.ds((i * _U + u) * 16, 16)] = zeros16
        return carry

    lax.fori_loop(0, _N // 16 // _U, zero, None)
    pltpu.sync_copy(ei_hbm.at[pl.ds(_E + gw * _EPW, _EPW)], dstv)
    ones16 = jnp.ones((16,), jnp.float32)

    def body(i, carry):
        for u in range(_U):
            idx = dstv[pl.ds((i * _U + u) * 16, 16)]
            plsc.addupdate_scatter(cnt, [idx], ones16)
        return carry

    lax.fori_loop(0, _EPW // 16 // _U, body, None)
    pltpu.sync_copy(cnt, out_hbm.at[pl.ds(gw * _N, _N)])


_deg = pl.kernel(
    _deg_body,
    out_type=jax.ShapeDtypeStruct((_NW * _N,), jnp.float32),
    mesh=_mesh(),
    compiler_params=pltpu.CompilerParams(needs_layout_passes=False),
    scratch_types=[
        pltpu.VMEM((_EPW,), jnp.int32),
        pltpu.VMEM((_N,), jnp.float32),
    ],
)


_R = 4


def _agg_body(ei_hbm, tab_hbm, out_hbm,
              srcall, rows0, rows1, rows2, rows3,
              dx0, dx1, dx2, dx3, drem, rrem,
              g0, g1, g2, g3, s0, s1, s2, s3,
              i0_, i1_, i2_, i3_, acc):
    rows = (rows0, rows1, rows2, rows3)
    dxs = (dx0, dx1, dx2, dx3)
    gsems = (g0, g1, g2, g3)
    ssems = (s0, s1, s2, s3)
    isems = (i0_, i1_, i2_, i3_)
    c = lax.axis_index("c")
    s = lax.axis_index("s")
    gw = c * _NS + s
    base_e = gw * _EPW
    row0 = s * _WSTRIDE

    zeros16 = jnp.zeros((16,), jnp.float32)

    def zrow(i, carry):
        r = i >> 3
        cc = (i & 7) * 16
        rows0[r, pl.ds(cc, 16)] = zeros16
        return carry

    lax.fori_loop(0, _CH * 8, zrow, None)
    nfull = _WSIZE // _CH
    ztail = _WSIZE - nfull * _CH
    for w in range(nfull):
        pltpu.async_copy(rows0, acc.at[pl.ds(row0 + w * _CH, _CH)], s0)
    if ztail:
        pltpu.async_copy(rows0.at[pl.ds(0, ztail)],
                         acc.at[pl.ds(row0 + nfull * _CH, ztail)], s0)
    pltpu.async_copy(ei_hbm.at[pl.ds(base_e, _EPW)], srcall, g0)

    def start_dx(i, r):
        pltpu.async_copy(ei_hbm.at[pl.ds(_E + base_e + i * _CH, _CH)],
                         dxs[r], isems[r])

    def wait_dx(r):
        pltpu.make_async_copy(ei_hbm.at[pl.ds(_E + base_e, _CH)],
                              dxs[r], isems[r]).wait()

    for r in range(_R):
        start_dx(r, r)
    for w in range(nfull):
        pltpu.make_async_copy(rows0, acc.at[pl.ds(row0 + w * _CH, _CH)],
                              s0).wait()
    if ztail:
        pltpu.make_async_copy(rows0.at[pl.ds(0, ztail)],
                              acc.at[pl.ds(row0 + nfull * _CH, ztail)],
                              s0).wait()
    pltpu.make_async_copy(ei_hbm.at[pl.ds(base_e, _EPW)], srcall, g0).wait()
    plsc.subcore_barrier()

    def start_gather(i, r):
        pltpu.async_copy(tab_hbm.at[srcall.at[pl.ds(i * _CH, _CH)]],
                         rows[r], gsems[r])

    def wait_gather(r):
        pltpu.make_async_copy(tab_hbm.at[srcall.at[pl.ds(0, _CH)]],
                              rows[r], gsems[r]).wait()

    def start_scatter(r):
        pltpu.async_copy(rows[r], acc.at[dxs[r]], ssems[r], add=True)

    def wait_scatter(r):
        pltpu.make_async_copy(rows[r], acc.at[dxs[r]], ssems[r]).wait()

    for r in range(_R):
        start_gather(r, r)

    def body(k, carry):
        i0 = k * _R
        for r in range(_R):
            wait_gather(r)
            wait_dx(r)
            start_scatter(r)
        for r in range(_R):
            wait_scatter(r)
            start_dx(i0 + _R + r, r)
            start_gather(i0 + _R + r, r)
        return carry

    lax.fori_loop(0, _NFULL // _R - 1, body, None)

    i0 = _NFULL - _R
    for r in range(_R):
        wait_gather(r)
        wait_dx(r)
        start_scatter(r)

    be = _NFULL * _CH
    pltpu.sync_copy(ei_hbm.at[pl.ds(_E + base_e + be, _REM)], drem)
    pltpu.sync_copy(tab_hbm.at[srcall.at[pl.ds(be, _REM)]], rrem)
    pltpu.sync_copy(rrem, acc.at[drem], add=True)

    for r in range(_R):
        wait_scatter(r)

    plsc.subcore_barrier()
    pltpu.sync_copy(acc.at[pl.ds(row0, _WSIZE)],
                    out_hbm.at[c, pl.ds(row0, _WSIZE)])


_agg = pl.kernel(
    _agg_body,
    out_type=jax.ShapeDtypeStruct((_NC, _N, _C), jnp.float32),
    mesh=_mesh(),
    compiler_params=pltpu.CompilerParams(needs_layout_passes=False),
    scratch_types=[
        pltpu.VMEM((_EPW,), jnp.int32),
        pltpu.VMEM((_CH, _C), jnp.float32),
        pltpu.VMEM((_CH, _C), jnp.float32),
        pltpu.VMEM((_CH, _C), jnp.float32),
        pltpu.VMEM((_CH, _C), jnp.float32),
        pltpu.VMEM((_CH,), jnp.int32),
        pltpu.VMEM((_CH,), jnp.int32),
        pltpu.VMEM((_CH,), jnp.int32),
        pltpu.VMEM((_CH,), jnp.int32),
        pltpu.VMEM((_REM,), jnp.int32),
        pltpu.VMEM((_REM, _C), jnp.float32),
        pltpu.SemaphoreType.DMA,
        pltpu.SemaphoreType.DMA,
        pltpu.SemaphoreType.DMA,
        pltpu.SemaphoreType.DMA,
        pltpu.SemaphoreType.DMA,
        pltpu.SemaphoreType.DMA,
        pltpu.SemaphoreType.DMA,
        pltpu.SemaphoreType.DMA,
        pltpu.SemaphoreType.DMA,
        pltpu.SemaphoreType.DMA,
        pltpu.SemaphoreType.DMA,
        pltpu.SemaphoreType.DMA,
        pltpu.VMEM_SHARED((_N, _C), jnp.float32),
    ],
)


def _bn(v, w, b):
    m = jnp.mean(v, axis=0, keepdims=True)
    d = v - m
    var = jnp.mean(d * d, axis=0, keepdims=True)
    return d * lax.rsqrt(var + _EPS) * w + b


def _k2a_body(x_ref, w0_ref, b0_ref, W1_ref, xw_ref):
    xn = _bn(x_ref[...], w0_ref[...], b0_ref[...])
    xw_ref[...] = jnp.dot(xn, W1_ref[...], preferred_element_type=jnp.float32)


_k2a = pl.pallas_call(
    _k2a_body,
    out_shape=jax.ShapeDtypeStruct((_N, _C), jnp.float32),
)


def _k2b_body(xw_ref, degp_ref, y_ref, dis_ref):
    ones = jnp.ones((_NW, 1), jnp.float32)
    deg = lax.dot_general(degp_ref[...], ones,
                          (((0,), (0,)), ((), ())),
                          preferred_element_type=jnp.float32) + 1.0
    dis = lax.rsqrt(deg)
    y_ref[...] = xw_ref[...] * dis
    dis_ref[...] = dis


_k2b = pl.pallas_call(
    _k2b_body,
    out_shape=(
        jax.ShapeDtypeStruct((_N, _C), jnp.float32),
        jax.ShapeDtypeStruct((_N, 1), jnp.float32),
    ),
)


def _k4_body(accp_ref, y_ref, dis_ref, b1_ref, w1_ref, bb1_ref, Wroot_ref,
             h_ref, hr_ref):
    acc = accp_ref[0] + accp_ref[1]
    g = dis_ref[...] * (acc + y_ref[...]) + b1_ref[...]
    g = jnp.maximum(g, 0.0)
    h = _bn(g, w1_ref[...], bb1_ref[...])
    h_ref[...] = h
    hr_ref[...] = jnp.dot(h, Wroot_ref[...],
                          preferred_element_type=jnp.float32)


_k4 = pl.pallas_call(
    _k4_body,
    out_shape=(
        jax.ShapeDtypeStruct((_N, _C), jnp.float32),
        jax.ShapeDtypeStruct((_N, _C), jnp.float32),
    ),
)


def _k6b_body(accp_ref, hr_ref, Wrel_ref, b2_ref, w2_ref, bb2_ref, o_ref):
    acc = accp_ref[0] + accp_ref[1]
    z = (jnp.dot(acc, Wrel_ref[...], preferred_element_type=jnp.float32)
         + b2_ref[...] + hr_ref[...])
    z = jnp.maximum(z, 0.0)
    o_ref[...] = _bn(z, w2_ref[...], bb2_ref[...])


_k6b = pl.pallas_call(
    _k6b_body,
    out_shape=jax.ShapeDtypeStruct((_N, _C), jnp.float32),
)


def kernel(x, edge_index, bn0_w, bn0_b, gcn1_W, gcn1_b, bn1_w, bn1_b,
           gc2_W_rel, gc2_W_root, gc2_b, bn2_w, bn2_b):
    ei = edge_index.astype(jnp.int32).reshape(2 * _E)

    degp = _deg(ei).reshape(_NW, _N)
    xw = _k2a(x, bn0_w.reshape(1, _C), bn0_b.reshape(1, _C), gcn1_W)
    y, dis = _k2b(xw, degp)
    accp = _agg(ei, y)
    h, hroot = _k4(accp, y, dis, gcn1_b.reshape(1, _C), bn1_w.reshape(1, _C),
                   bn1_b.reshape(1, _C), gc2_W_root)
    acc2p = _agg(ei, h)
    out = _k6b(acc2p, hroot, gc2_W_rel, gc2_b.reshape(1, _C),
               bn2_w.reshape(1, _C), bn2_b.reshape(1, _C))
    return out


# speedup vs baseline: 1.0257x; 1.0022x over previous
import functools

import jax
import jax.numpy as jnp
from jax import lax
from jax.experimental import pallas as pl
from jax.experimental.pallas import tpu as pltpu
from jax.experimental.pallas import tpu_sc as plsc

_N = 10000
_E = 320000
_C = 128
_EPS = 1e-5

_NC = 2
_NS = 16
_NW = _NC * _NS
_EPW = _E // _NW
_RPT = _N // _NS
_CH = 40
_NFULL = _EPW // _CH
_REM = _EPW - _NFULL * _CH

_WSTRIDE = 624
_WSIZE = 640


def _mesh():
    return plsc.VectorSubcoreMesh(core_axis_name="c", subcore_axis_name="s")


def _deg_body(ei_hbm, out_hbm, dstv, cnt):
    c = lax.axis_index("c")
    s = lax.axis_index("s")
    gw = c * _NS + s
    zeros16 = jnp.zeros((16,), jnp.float32)
    _U = 5

    def zero(i, carry):
        for u in range(_U):
            cnt[pl.ds((i * _U + u) * 16, 16)] = zeros16
        return carry

    lax.fori_loop(0, _N // 16 // _U, zero, None)
    pltpu.sync_copy(ei_hbm.at[pl.ds(_E + gw * _EPW, _EPW)], dstv)
    ones16 = jnp.ones((16,), jnp.float32)

    def body(i, carry):
        for u in range(_U):
            idx = dstv[pl.ds((i * _U + u) * 16, 16)]
            plsc.addupdate_scatter(cnt, [idx], ones16)
        return carry

    lax.fori_loop(0, _EPW // 16 // _U, body, None)
    pltpu.sync_copy(cnt, out_hbm.at[pl.ds(gw * _N, _N)])


_deg = pl.kernel(
    _deg_body,
    out_type=jax.ShapeDtypeStruct((_NW * _N,), jnp.float32),
    mesh=_mesh(),
    compiler_params=pltpu.CompilerParams(needs_layout_passes=False),
    scratch_types=[
        pltpu.VMEM((_EPW,), jnp.int32),
        pltpu.VMEM((_N,), jnp.float32),
    ],
)


_R = 5


def _agg_body(ei_hbm, tab_hbm, out_hbm,
              srcall, rows0, rows1, rows2, rows3, rows4,
              dx0, dx1, dx2, dx3, dx4,
              g0, g1, g2, g3, g4, s0, s1, s2, s3, s4,
              i0_, i1_, i2_, i3_, i4_, acc):
    rows = (rows0, rows1, rows2, rows3, rows4)
    dxs = (dx0, dx1, dx2, dx3, dx4)
    gsems = (g0, g1, g2, g3, g4)
    ssems = (s0, s1, s2, s3, s4)
    isems = (i0_, i1_, i2_, i3_, i4_)
    c = lax.axis_index("c")
    s = lax.axis_index("s")
    gw = c * _NS + s
    base_e = gw * _EPW
    row0 = s * _WSTRIDE

    zeros16 = jnp.zeros((16,), jnp.float32)

    def zrow(i, carry):
        r = i >> 3
        cc = (i & 7) * 16
        rows0[r, pl.ds(cc, 16)] = zeros16
        return carry

    lax.fori_loop(0, _CH * 8, zrow, None)
    nfull = _WSIZE // _CH
    ztail = _WSIZE - nfull * _CH
    for w in range(nfull):
        pltpu.async_copy(rows0, acc.at[pl.ds(row0 + w * _CH, _CH)], s0)
    if ztail:
        pltpu.async_copy(rows0.at[pl.ds(0, ztail)],
                         acc.at[pl.ds(row0 + nfull * _CH, ztail)], s0)
    pltpu.async_copy(ei_hbm.at[pl.ds(base_e, _EPW)], srcall, g0)

    def start_dx(i, r):
        pltpu.async_copy(ei_hbm.at[pl.ds(_E + base_e + i * _CH, _CH)],
                         dxs[r], isems[r])

    def wait_dx(r):
        pltpu.make_async_copy(ei_hbm.at[pl.ds(_E + base_e, _CH)],
                              dxs[r], isems[r]).wait()

    for r in range(_R):
        start_dx(r, r)
    for w in range(nfull):
        pltpu.make_async_copy(rows0, acc.at[pl.ds(row0 + w * _CH, _CH)],
                              s0).wait()
    if ztail:
        pltpu.make_async_copy(rows0.at[pl.ds(0, ztail)],
                              acc.at[pl.ds(row0 + nfull * _CH, ztail)],
                              s0).wait()
    pltpu.make_async_copy(ei_hbm.at[pl.ds(base_e, _EPW)], srcall, g0).wait()
    plsc.subcore_barrier()

    def start_gather(i, r):
        pltpu.async_copy(tab_hbm.at[srcall.at[pl.ds(i * _CH, _CH)]],
                         rows[r], gsems[r])

    def wait_gather(r):
        pltpu.make_async_copy(tab_hbm.at[srcall.at[pl.ds(0, _CH)]],
                              rows[r], gsems[r]).wait()

    def start_scatter(r):
        pltpu.async_copy(rows[r], acc.at[dxs[r]], ssems[r], add=True)

    def wait_scatter(r):
        pltpu.make_async_copy(rows[r], acc.at[dxs[r]], ssems[r]).wait()

    for r in range(_R):
        start_gather(r, r)

    def body(k, carry):
        i0 = k * _R
        for r in range(_R):
            wait_gather(r)
            wait_dx(r)
            start_scatter(r)
        for r in range(_R):
            wait_scatter(r)
            start_dx(i0 + _R + r, r)
            start_gather(i0 + _R + r, r)
        return carry

    lax.fori_loop(0, _NFULL // _R - 1, body, None)

    for r in range(_R):
        wait_gather(r)
        wait_dx(r)
        start_scatter(r)
    for r in range(_R):
        wait_scatter(r)

    plsc.subcore_barrier()
    pltpu.sync_copy(acc.at[pl.ds(row0, _WSIZE)],
                    out_hbm.at[c, pl.ds(row0, _WSIZE)])


_agg = pl.kernel(
    _agg_body,
    out_type=jax.ShapeDtypeStruct((_NC, _N, _C), jnp.float32),
    mesh=_mesh(),
    compiler_params=pltpu.CompilerParams(needs_layout_passes=False),
    scratch_types=[
        pltpu.VMEM((_EPW,), jnp.int32),
        pltpu.VMEM((_CH, _C), jnp.float32),
        pltpu.VMEM((_CH, _C), jnp.float32),
        pltpu.VMEM((_CH, _C), jnp.float32),
        pltpu.VMEM((_CH, _C), jnp.float32),
        pltpu.VMEM((_CH, _C), jnp.float32),
        pltpu.VMEM((_CH,), jnp.int32),
        pltpu.VMEM((_CH,), jnp.int32),
        pltpu.VMEM((_CH,), jnp.int32),
        pltpu.VMEM((_CH,), jnp.int32),
        pltpu.VMEM((_CH,), jnp.int32),
        pltpu.SemaphoreType.DMA,
        pltpu.SemaphoreType.DMA,
        pltpu.SemaphoreType.DMA,
        pltpu.SemaphoreType.DMA,
        pltpu.SemaphoreType.DMA,
        pltpu.SemaphoreType.DMA,
        pltpu.SemaphoreType.DMA,
        pltpu.SemaphoreType.DMA,
        pltpu.SemaphoreType.DMA,
        pltpu.SemaphoreType.DMA,
        pltpu.SemaphoreType.DMA,
        pltpu.SemaphoreType.DMA,
        pltpu.SemaphoreType.DMA,
        pltpu.SemaphoreType.DMA,
        pltpu.SemaphoreType.DMA,
        pltpu.VMEM_SHARED((_N, _C), jnp.float32),
    ],
)


def _bn(v, w, b):
    m = jnp.mean(v, axis=0, keepdims=True)
    d = v - m
    var = jnp.mean(d * d, axis=0, keepdims=True)
    return d * lax.rsqrt(var + _EPS) * w + b


def _k2a_body(x_ref, w0_ref, b0_ref, W1_ref, xw_ref):
    xn = _bn(x_ref[...], w0_ref[...], b0_ref[...])
    xw_ref[...] = jnp.dot(xn, W1_ref[...], preferred_element_type=jnp.float32)


_k2a = pl.pallas_call(
    _k2a_body,
    out_shape=jax.ShapeDtypeStruct((_N, _C), jnp.float32),
)


def _k2b_body(xw_ref, degp_ref, y_ref, dis_ref):
    ones = jnp.ones((_NW, 1), jnp.float32)
    deg = lax.dot_general(degp_ref[...], ones,
                          (((0,), (0,)), ((), ())),
                          preferred_element_type=jnp.float32) + 1.0
    dis = lax.rsqrt(deg)
    y_ref[...] = xw_ref[...] * dis
    dis_ref[...] = dis


_k2b = pl.pallas_call(
    _k2b_body,
    out_shape=(
        jax.ShapeDtypeStruct((_N, _C), jnp.float32),
        jax.ShapeDtypeStruct((_N, 1), jnp.float32),
    ),
)


def _k4_body(accp_ref, y_ref, dis_ref, b1_ref, w1_ref, bb1_ref, Wroot_ref,
             h_ref, hr_ref):
    acc = accp_ref[0] + accp_ref[1]
    g = dis_ref[...] * (acc + y_ref[...]) + b1_ref[...]
    g = jnp.maximum(g, 0.0)
    h = _bn(g, w1_ref[...], bb1_ref[...])
    h_ref[...] = h
    hr_ref[...] = jnp.dot(h, Wroot_ref[...],
                          preferred_element_type=jnp.float32)


_k4 = pl.pallas_call(
    _k4_body,
    out_shape=(
        jax.ShapeDtypeStruct((_N, _C), jnp.float32),
        jax.ShapeDtypeStruct((_N, _C), jnp.float32),
    ),
)


def _k6b_body(accp_ref, hr_ref, Wrel_ref, b2_ref, w2_ref, bb2_ref, o_ref):
    acc = accp_ref[0] + accp_ref[1]
    z = (jnp.dot(acc, Wrel_ref[...], preferred_element_type=jnp.float32)
         + b2_ref[...] + hr_ref[...])
    z = jnp.maximum(z, 0.0)
    o_ref[...] = _bn(z, w2_ref[...], bb2_ref[...])


_k6b = pl.pallas_call(
    _k6b_body,
    out_shape=jax.ShapeDtypeStruct((_N, _C), jnp.float32),
)


def kernel(x, edge_index, bn0_w, bn0_b, gcn1_W, gcn1_b, bn1_w, bn1_b,
           gc2_W_rel, gc2_W_root, gc2_b, bn2_w, bn2_b):
    ei = edge_index.astype(jnp.int32).reshape(2 * _E)

    degp = _deg(ei).reshape(_NW, _N)
    xw = _k2a(x, bn0_w.reshape(1, _C), bn0_b.reshape(1, _C), gcn1_W)
    y, dis = _k2b(xw, degp)
    accp = _agg(ei, y)
    h, hroot = _k4(accp, y, dis, gcn1_b.reshape(1, _C), bn1_w.reshape(1, _C),
                   bn1_b.reshape(1, _C), gc2_W_root)
    acc2p = _agg(ei, h)
    out = _k6b(acc2p, hroot, gc2_W_rel, gc2_b.reshape(1, _C),
               bn2_w.reshape(1, _C), bn2_b.reshape(1, _C))
    return out
